# Initial kernel scaffold; baseline (speedup 1.0000x reference)
import jax
import jax.numpy as jnp
from jax.experimental import pallas as pl


def kernel(positions, densities, grid_ema, aabb_min, aabb_max):
    raise NotImplementedError("write your pallas kernel here")


# speedup vs baseline: 2.6624x; 2.6624x over previous
import jax
import jax.numpy as jnp
from jax import lax
from jax.experimental import pallas as pl
from jax.experimental.pallas import tpu as pltpu
from jax.experimental.pallas import tpu_sc as plsc

RES = 128
EMA_TAU = 0.95
NVOX = RES * RES * RES
NS = 16
NP = 1 << 20
CHUNK = NP // NS
SUB = CHUNK // 4


def _idx_body(pos_ref, amin_ref, amax_ref, idx_ref):
    pos = pos_ref[...]
    amin = amin_ref[...]
    amax = amax_ref[...]
    xn = (pos - amin) / (amax - amin)
    xn = jnp.clip(xn, 0.0, 1.0)
    c = jnp.clip((xn * RES).astype(jnp.int32), 0, RES - 1)
    idx = c[0:1, :] * (RES * RES) + c[1:2, :] * RES + c[2:3, :]
    idx_ref[...] = idx


def _compute_idx(pos_t, amin, amax):
    B = 1 << 16
    nblk = NP // B
    return pl.pallas_call(
        _idx_body,
        grid=(nblk,),
        in_specs=[
            pl.BlockSpec((3, B), lambda i: (0, i)),
            pl.BlockSpec((3, 1), lambda i: (0, 0)),
            pl.BlockSpec((3, 1), lambda i: (0, 0)),
        ],
        out_specs=pl.BlockSpec((1, B), lambda i: (0, i)),
        out_shape=jax.ShapeDtypeStruct((1, NP), jnp.int32),
    )(pos_t, amin, amax)


NH = NVOX // 2
TRASH = NH
GS = NH // NS


def _transform_idx(idx_v, base_vox):
    trash = jnp.full((16,), TRASH, jnp.uint32)

    def body(k, _):
        start = pl.multiple_of(k * 16, 16)
        iv = idx_v[pl.ds(start, 16)]
        loc = plsc.bitcast(iv - base_vox, jnp.uint32)
        idx_v[pl.ds(start, 16)] = plsc.bitcast(jnp.minimum(loc, trash),
                                               jnp.int32)
        return 0

    lax.fori_loop(0, SUB // 16, body, 0)


def _sc_body(idx_hbm, val_hbm, zeros_hbm, sum_hbm, cnt_hbm, idx_v, val_v, shared):
    cid = lax.axis_index("c")
    sid = lax.axis_index("s")
    base_vox = cid * NH
    gbase = sid * GS
    base = sid * CHUNK

    for p, out_hbm in ((0, sum_hbm), (1, cnt_hbm)):
        pltpu.sync_copy(zeros_hbm.at[pl.ds(gbase, GS)],
                        shared.at[pl.ds(gbase, GS)])
        plsc.subcore_barrier()

        for j in range(CHUNK // SUB):
            off = base + j * SUB
            pltpu.sync_copy(idx_hbm.at[pl.ds(off, SUB)], idx_v)
            _transform_idx(idx_v, base_vox)
            pltpu.sync_copy(val_hbm.at[p, pl.ds(off, SUB)], val_v)
            pltpu.sync_copy(val_v, shared.at[idx_v], add=True)
        plsc.subcore_barrier()

        pltpu.sync_copy(shared.at[pl.ds(gbase, GS)],
                        out_hbm.at[pl.ds(base_vox + gbase, GS)])


_sc_scatter = pl.kernel(
    _sc_body,
    out_type=(
        jax.ShapeDtypeStruct((NVOX,), jnp.float32),
        jax.ShapeDtypeStruct((NVOX,), jnp.float32),
    ),
    mesh=plsc.VectorSubcoreMesh(core_axis_name="c", subcore_axis_name="s"),
    scratch_types=[
        pltpu.VMEM((SUB,), jnp.int32),
        pltpu.VMEM((SUB,), jnp.float32),
        pltpu.VMEM_SHARED((NH + 8,), jnp.float32),
    ],
)


def _ema_body(ema_ref, sum_ref, cnt_ref, out_ref):
    ema = ema_ref[...]
    s = sum_ref[...]
    c = cnt_ref[...]
    mean = jnp.where(c > 0.0, s / jnp.where(c > 0.0, c, 1.0), 0.0)
    out_ref[...] = EMA_TAU * ema + (1.0 - EMA_TAU) * mean


def _ema_combine(ema, gsum, gcnt):
    R, C = 2048, 1024
    BR = 256
    return pl.pallas_call(
        _ema_body,
        grid=(R // BR,),
        in_specs=[pl.BlockSpec((BR, C), lambda i: (i, 0))] * 3,
        out_specs=pl.BlockSpec((BR, C), lambda i: (i, 0)),
        out_shape=jax.ShapeDtypeStruct((R, C), jnp.float32),
    )(ema.reshape(R, C), gsum.reshape(R, C), gcnt.reshape(R, C))


@jax.jit
def kernel(positions, densities, grid_ema, aabb_min, aabb_max):
    n = positions.shape[0]
    pos_t = jnp.pad(positions, ((0, NP - n), (0, 0))).T
    idx = _compute_idx(pos_t, aabb_min.reshape(3, 1), aabb_max.reshape(3, 1))
    idx = idx.reshape(NP)
    dens_p = jnp.pad(densities, (0, NP - n))
    ones_p = jnp.pad(jnp.ones((n,), jnp.float32), (0, NP - n))
    vals2 = jnp.stack([dens_p, ones_p])
    zeros = jnp.zeros((NH,), jnp.float32)
    gsum, gcnt = _sc_scatter(idx, vals2, zeros)
    out = _ema_combine(grid_ema.reshape(-1), gsum, gcnt)
    return out.reshape(RES, RES, RES)


# speedup vs baseline: 2.6897x; 1.0103x over previous
import jax
import jax.numpy as jnp
from jax import lax
from jax.experimental import pallas as pl
from jax.experimental.pallas import tpu as pltpu
from jax.experimental.pallas import tpu_sc as plsc

RES = 128
EMA_TAU = 0.95
NVOX = RES * RES * RES
NS = 16
NP = 1 << 20
CHUNK = NP // NS
SUB = CHUNK // 4


HUGE = 1 << 30


def _idx_body(n_real, blk, pos_ref, amin_ref, amax_ref, idx_ref):
    pos = pos_ref[...]
    amin = amin_ref[...]
    amax = amax_ref[...]
    xn = (pos - amin) / (amax - amin)
    xn = jnp.clip(xn, 0.0, 1.0)
    c = jnp.clip((xn * RES).astype(jnp.int32), 0, RES - 1)
    idx = c[0:1, :] * (RES * RES) + c[1:2, :] * RES + c[2:3, :]
    lane = jax.lax.broadcasted_iota(jnp.int32, idx.shape, 1) \
        + pl.program_id(0) * blk
    idx_ref[...] = jnp.where(lane < n_real, idx, HUGE)


def _compute_idx(pos_t, amin, amax, n_real):
    B = 1 << 16
    nblk = NP // B
    return pl.pallas_call(
        lambda *refs: _idx_body(n_real, B, *refs),
        grid=(nblk,),
        in_specs=[
            pl.BlockSpec((3, B), lambda i: (0, i)),
            pl.BlockSpec((3, 1), lambda i: (0, 0)),
            pl.BlockSpec((3, 1), lambda i: (0, 0)),
        ],
        out_specs=pl.BlockSpec((1, B), lambda i: (0, i)),
        out_shape=jax.ShapeDtypeStruct((1, NP), jnp.int32),
    )(pos_t, amin, amax)


NH = NVOX // 2
TRASH = NH
GS = NH // NS


def _transform_idx(idx_v, base_vox):
    trash = jnp.full((16,), TRASH, jnp.uint32)

    def body(k, _):
        for u in range(4):
            start = pl.multiple_of(k * 64 + u * 16, 16)
            iv = idx_v[pl.ds(start, 16)]
            loc = plsc.bitcast(iv - base_vox, jnp.uint32)
            idx_v[pl.ds(start, 16)] = plsc.bitcast(jnp.minimum(loc, trash),
                                                   jnp.int32)
        return 0

    lax.fori_loop(0, SUB // 64, body, 0)


def _sc_body(idx_hbm, dens_hbm, ones_hbm, zeros_hbm, sum_hbm, cnt_hbm,
             idx_v, val_v, ones_v, shared):
    cid = lax.axis_index("c")
    sid = lax.axis_index("s")
    base_vox = cid * NH
    gbase = sid * GS
    base = sid * CHUNK

    pltpu.sync_copy(ones_hbm, ones_v)

    for p, out_hbm in ((0, sum_hbm), (1, cnt_hbm)):
        pltpu.sync_copy(zeros_hbm.at[pl.ds(gbase, GS)],
                        shared.at[pl.ds(gbase, GS)])
        plsc.subcore_barrier()

        for j in range(CHUNK // SUB):
            off = base + j * SUB
            pltpu.sync_copy(idx_hbm.at[pl.ds(off, SUB)], idx_v)
            _transform_idx(idx_v, base_vox)
            if p == 0:
                pltpu.sync_copy(dens_hbm.at[pl.ds(off, SUB)], val_v)
                pltpu.sync_copy(val_v, shared.at[idx_v], add=True)
            else:
                pltpu.sync_copy(ones_v, shared.at[idx_v], add=True)
        plsc.subcore_barrier()

        pltpu.sync_copy(shared.at[pl.ds(gbase, GS)],
                        out_hbm.at[pl.ds(base_vox + gbase, GS)])


_sc_scatter = pl.kernel(
    _sc_body,
    out_type=(
        jax.ShapeDtypeStruct((NVOX,), jnp.float32),
        jax.ShapeDtypeStruct((NVOX,), jnp.float32),
    ),
    mesh=plsc.VectorSubcoreMesh(core_axis_name="c", subcore_axis_name="s"),
    scratch_types=[
        pltpu.VMEM((SUB,), jnp.int32),
        pltpu.VMEM((SUB,), jnp.float32),
        pltpu.VMEM((SUB,), jnp.float32),
        pltpu.VMEM_SHARED((NH + 8,), jnp.float32),
    ],
)


def _ema_body(ema_ref, sum_ref, cnt_ref, out_ref):
    ema = ema_ref[...]
    s = sum_ref[...]
    c = cnt_ref[...]
    mean = jnp.where(c > 0.0, s / jnp.where(c > 0.0, c, 1.0), 0.0)
    out_ref[...] = EMA_TAU * ema + (1.0 - EMA_TAU) * mean


def _ema_combine(ema, gsum, gcnt):
    R, C = 2048, 1024
    BR = 256
    return pl.pallas_call(
        _ema_body,
        grid=(R // BR,),
        in_specs=[pl.BlockSpec((BR, C), lambda i: (i, 0))] * 3,
        out_specs=pl.BlockSpec((BR, C), lambda i: (i, 0)),
        out_shape=jax.ShapeDtypeStruct((R, C), jnp.float32),
    )(ema.reshape(R, C), gsum.reshape(R, C), gcnt.reshape(R, C))


@jax.jit
def kernel(positions, densities, grid_ema, aabb_min, aabb_max):
    n = positions.shape[0]
    pos_t = jnp.pad(positions, ((0, NP - n), (0, 0))).T
    idx = _compute_idx(pos_t, aabb_min.reshape(3, 1),
                       aabb_max.reshape(3, 1), n)
    idx = idx.reshape(NP)
    dens_p = jnp.pad(densities, (0, NP - n))
    ones = jnp.ones((SUB,), jnp.float32)
    zeros = jnp.zeros((NH,), jnp.float32)
    gsum, gcnt = _sc_scatter(idx, dens_p, ones, zeros)
    out = _ema_combine(grid_ema.reshape(-1), gsum, gcnt)
    return out.reshape(RES, RES, RES)
